# baseline (speedup 1.0000x reference)
import functools
import math

import jax
import jax.numpy as jnp
from jax import lax
from jax.experimental import pallas as pl
from jax.experimental.pallas import tpu as pltpu
from jax.experimental.pallas import tpu_sc as plsc

N = 131072
H = 16384
P = 8
D = 128
HID = 512
TT = 64
S = P + 1

NC = 2
NS = 16
NW = NC * NS
HCH = 8
RCH = S * HCH
NSPLIT = 4
NBUF = 4
LANES = 16
NJ = D // LANES


def _sc_pool(hf, qs, pi_t, po, val_t, h_off, nh):
    HPW = nh // NW
    RPW = S * HPW
    NCH = HPW // HCH
    mesh = plsc.VectorSubcoreMesh(core_axis_name="c", subcore_axis_name="s")

    @functools.partial(
        pl.kernel,
        mesh=mesh,
        compiler_params=pltpu.CompilerParams(needs_layout_passes=False),
        out_type=jax.ShapeDtypeStruct((nh, D), jnp.float32),
        scratch_types=[
            pltpu.VMEM((S * HPW,), jnp.int32),
            pltpu.VMEM((P * HPW,), jnp.float32),
            pltpu.VMEM((RPW,), jnp.int32),
            pltpu.VMEM((D,), jnp.float32),
            pltpu.VMEM((RPW + LANES,), jnp.float32),
            pltpu.VMEM((NBUF, RCH, D), jnp.float32),
            pltpu.VMEM((NBUF, HCH, D), jnp.float32),
            pltpu.SemaphoreType.DMA,
            pltpu.SemaphoreType.DMA((NBUF,)),
            pltpu.SemaphoreType.DMA((NBUF,)),
        ],
    )
    def k(hf_hbm, q_hbm, pit_hbm, po_hbm, valt_hbm, out_hbm,
          sm_v, vsm_v, ihm_v, q_v, val_v, rows_v, outb_v, psem, gsem, osem):
        wid = lax.axis_index("s") * NC + lax.axis_index("c")
        cps = []
        for s in range(P):
            cps.append(pltpu.make_async_copy(
                pit_hbm.at[s, pl.ds(h_off + wid * HPW, HPW)],
                sm_v.at[pl.ds(s * HPW, HPW)], psem))
            cps.append(pltpu.make_async_copy(
                valt_hbm.at[s, pl.ds(h_off + wid * HPW, HPW)],
                vsm_v.at[pl.ds(s * HPW, HPW)], psem))
        cps.append(pltpu.make_async_copy(
            po_hbm.at[pl.ds(h_off + wid * HPW, HPW)],
            sm_v.at[pl.ds(P * HPW, HPW)], psem))
        cps.append(pltpu.make_async_copy(q_hbm, q_v, psem))
        for cp in cps:
            cp.start()
        for cp in cps:
            cp.wait()

        def build(g, _):
            h0 = g * LANES
            base = lax.broadcasted_iota(jnp.int32, (LANES,), 0) * S + h0 * S
            for s in range(S):
                plsc.store_scatter(ihm_v, [base + s],
                                   sm_v[pl.ds(s * HPW + h0, LANES)])
            for s in range(P):
                plsc.store_scatter(val_v, [base + s],
                                   vsm_v[pl.ds(s * HPW + h0, LANES)])
            return 0

        lax.fori_loop(0, HPW // LANES, build, 0)

        def gather(c, b):
            return pltpu.make_async_copy(
                hf_hbm.at[ihm_v.at[pl.ds(c * RCH, RCH)]], rows_v.at[b],
                gsem.at[b])

        def outcp(c, b):
            return pltpu.make_async_copy(
                outb_v.at[b],
                out_hbm.at[pl.ds((wid * NCH + c) * HCH, HCH)],
                osem.at[b])

        for b in range(NBUF):
            gather(b, b).start()

        qr = [q_v[pl.ds(j * LANES, LANES)] for j in range(NJ)]

        def group(g, _):
            for b in range(NBUF):
                c = g * NBUF + b
                gather(c, b).wait()

                @pl.when(g > 0)
                def _():
                    outcp(c - NBUF, b).wait()

                def pool_hop(hh, _c):
                    rb = rows_v.at[b]
                    vv = val_v[pl.ds(_c * RCH + hh * S, LANES)]
                    accs = [jnp.zeros((LANES,), jnp.float32)
                            for _ in range(NJ)]
                    denom = jnp.zeros((LANES,), jnp.float32)
                    for sg in range(S // 3):
                        rows3 = []
                        psum3 = []
                        for k in range(3):
                            s = sg * 3 + k
                            row = [rb[hh * S + s, pl.ds(j * LANES, LANES)]
                                   for j in range(NJ)]
                            p = [row[j] * qr[j] for j in range(NJ)]
                            p = [p[0] + p[1], p[2] + p[3],
                                 p[4] + p[5], p[6] + p[7]]
                            p = [p[0] + p[1], p[2] + p[3]]
                            rows3.append(row)
                            psum3.append(p[0] + p[1])
                        tot3 = [plsc.cumsum(ps)[LANES - 1] for ps in psum3]
                        for k in range(3):
                            s = sg * 3 + k
                            e = jnp.exp(jnp.broadcast_to(tot3[k], (LANES,)))
                            if s < S - 1:
                                e = e * jnp.broadcast_to(vv[s], (LANES,))
                            row = rows3[k]
                            accs = [accs[j] + e * row[j] for j in range(NJ)]
                            denom = denom + e
                    rv = 1.0 / denom
                    for j in range(NJ):
                        outb_v[b, hh, pl.ds(j * LANES, LANES)] = accs[j] * rv

                def pool_one(hh, _c):
                    pool_hop(hh, _c)
                    return _c

                lax.fori_loop(0, HCH, pool_one, c)
                outcp(c, b).start()

                @pl.when(g < NCH // NBUF - 1)
                def _():
                    gather(c + NBUF, b).start()
            return 0

        lax.fori_loop(0, NCH // NBUF, group, 0)
        for b in range(NBUF):
            outcp(NCH - NBUF + b, b).wait()

    return k(hf, qs, pi_t, po, val_t)


BH = 1024


def _mlp_body(hop_ref, W1_ref, b1_ref, W2g_ref, g2_ref, c2_ref, out_ref):
    x = hop_ref[...]
    h = jnp.dot(x.astype(jnp.bfloat16), W1_ref[...],
                preferred_element_type=jnp.float32)
    h = jnp.maximum(h + b1_ref[...], 0.0)
    mu = jnp.mean(h, axis=-1, keepdims=True)
    m2 = jnp.mean(h * h, axis=-1, keepdims=True)
    rs = lax.rsqrt(m2 - mu * mu + 1e-5)
    z = jnp.dot(h.astype(jnp.bfloat16), W2g_ref[...],
                preferred_element_type=jnp.float32)
    res = rs * z - (rs * mu) * g2_ref[...] + c2_ref[...]
    out_ref[...] = res.T


def _tc_mlp(hop_hf, W1, b1, W2g, g2, c2):
    nh = hop_hf.shape[0]
    return pl.pallas_call(
        _mlp_body,
        grid=(nh // BH,),
        in_specs=[
            pl.BlockSpec((BH, D), lambda i: (i, 0)),
            pl.BlockSpec((D, HID), lambda i: (0, 0)),
            pl.BlockSpec((1, HID), lambda i: (0, 0)),
            pl.BlockSpec((HID, TT), lambda i: (0, 0)),
            pl.BlockSpec((1, TT), lambda i: (0, 0)),
            pl.BlockSpec((1, TT), lambda i: (0, 0)),
        ],
        out_specs=pl.BlockSpec((TT, BH), lambda i: (0, i)),
        out_shape=jax.ShapeDtypeStruct((TT, nh), jnp.float32),
    )(hop_hf, W1, b1, W2g, g2, c2)


def kernel(hf, q, W1, b1, gamma, beta, W2, b2, all_hop_pi, all_hop_pi_stats,
           all_hop_po):
    pi_t = all_hop_pi.T
    val_t = (all_hop_pi_stats != -1).astype(jnp.float32).T

    qs = q * (1.0 / math.sqrt(D))
    W2g = (gamma[:, None] * W2).astype(jnp.bfloat16)
    g2 = (gamma @ W2).reshape(1, TT)
    c2 = (beta @ W2 + b2).reshape(1, TT)
    W1b = W1.astype(jnp.bfloat16)
    b1r = b1.reshape(1, HID)
    nh = H // NSPLIT
    outs = []
    for i in range(NSPLIT):
        hop_hf = _sc_pool(hf, qs, pi_t, all_hop_po, val_t, i * nh, nh)
        outs.append(_tc_mlp(hop_hf, W1b, b1r, W2g, g2, c2))
    return jnp.concatenate(outs, axis=1).T


# speedup vs baseline: 1.1659x; 1.1659x over previous
import functools
import math

import jax
import jax.numpy as jnp
from jax import lax
from jax.experimental import pallas as pl
from jax.experimental.pallas import tpu as pltpu
from jax.experimental.pallas import tpu_sc as plsc

N = 131072
H = 16384
P = 8
D = 128
HID = 512
TT = 64
S = P + 1

NC = 2
NS = 16
NW = NC * NS
HCH = 8
RCH = S * HCH
NSPLIT = 2
NBUF = 4
LANES = 16
NJ = D // LANES


def _sc_pool(hf, qs, pi_t, po, val_t, h_off, nh):
    HPW = nh // NW
    RPW = S * HPW
    NCH = HPW // HCH
    mesh = plsc.VectorSubcoreMesh(core_axis_name="c", subcore_axis_name="s")

    @functools.partial(
        pl.kernel,
        mesh=mesh,
        compiler_params=pltpu.CompilerParams(needs_layout_passes=False),
        out_type=jax.ShapeDtypeStruct((nh, D), jnp.float32),
        scratch_types=[
            pltpu.VMEM((S * HPW,), jnp.int32),
            pltpu.VMEM((P * HPW,), jnp.float32),
            pltpu.VMEM((RPW,), jnp.int32),
            pltpu.VMEM((D,), jnp.float32),
            pltpu.VMEM((RPW + LANES,), jnp.float32),
            pltpu.VMEM((NBUF, RCH, D), jnp.float32),
            pltpu.VMEM((NBUF, HCH, D), jnp.float32),
            pltpu.SemaphoreType.DMA,
            pltpu.SemaphoreType.DMA((NBUF,)),
            pltpu.SemaphoreType.DMA((NBUF,)),
        ],
    )
    def k(hf_hbm, q_hbm, pit_hbm, po_hbm, valt_hbm, out_hbm,
          sm_v, vsm_v, ihm_v, q_v, val_v, rows_v, outb_v, psem, gsem, osem):
        wid = lax.axis_index("s") * NC + lax.axis_index("c")
        cps = []
        for s in range(P):
            cps.append(pltpu.make_async_copy(
                pit_hbm.at[s, pl.ds(h_off + wid * HPW, HPW)],
                sm_v.at[pl.ds(s * HPW, HPW)], psem))
            cps.append(pltpu.make_async_copy(
                valt_hbm.at[s, pl.ds(h_off + wid * HPW, HPW)],
                vsm_v.at[pl.ds(s * HPW, HPW)], psem))
        cps.append(pltpu.make_async_copy(
            po_hbm.at[pl.ds(h_off + wid * HPW, HPW)],
            sm_v.at[pl.ds(P * HPW, HPW)], psem))
        cps.append(pltpu.make_async_copy(q_hbm, q_v, psem))
        for cp in cps:
            cp.start()
        for cp in cps:
            cp.wait()

        def build(g, _):
            h0 = g * LANES
            base = lax.broadcasted_iota(jnp.int32, (LANES,), 0) * S + h0 * S
            for s in range(S):
                plsc.store_scatter(ihm_v, [base + s],
                                   sm_v[pl.ds(s * HPW + h0, LANES)])
            for s in range(P):
                plsc.store_scatter(val_v, [base + s],
                                   vsm_v[pl.ds(s * HPW + h0, LANES)])
            return 0

        lax.fori_loop(0, HPW // LANES, build, 0)

        def gather(c, b):
            return pltpu.make_async_copy(
                hf_hbm.at[ihm_v.at[pl.ds(c * RCH, RCH)]], rows_v.at[b],
                gsem.at[b])

        def outcp(c, b):
            return pltpu.make_async_copy(
                outb_v.at[b],
                out_hbm.at[pl.ds((wid * NCH + c) * HCH, HCH)],
                osem.at[b])

        for b in range(NBUF):
            gather(b, b).start()

        qr = [q_v[pl.ds(j * LANES, LANES)] for j in range(NJ)]

        def group(g, _):
            for b in range(NBUF):
                c = g * NBUF + b
                gather(c, b).wait()

                @pl.when(g > 0)
                def _():
                    outcp(c - NBUF, b).wait()

                def pool_hop(hh, _c):
                    rb = rows_v.at[b]
                    vv = val_v[pl.ds(_c * RCH + hh * S, LANES)]
                    accs = [jnp.zeros((LANES,), jnp.float32)
                            for _ in range(NJ)]
                    denom = jnp.zeros((LANES,), jnp.float32)
                    for sg in range(S // 3):
                        rows3 = []
                        psum3 = []
                        for k in range(3):
                            s = sg * 3 + k
                            row = [rb[hh * S + s, pl.ds(j * LANES, LANES)]
                                   for j in range(NJ)]
                            p = [row[j] * qr[j] for j in range(NJ)]
                            p = [p[0] + p[1], p[2] + p[3],
                                 p[4] + p[5], p[6] + p[7]]
                            p = [p[0] + p[1], p[2] + p[3]]
                            rows3.append(row)
                            psum3.append(p[0] + p[1])
                        tot3 = [plsc.cumsum(ps)[LANES - 1] for ps in psum3]
                        for k in range(3):
                            s = sg * 3 + k
                            e = jnp.exp(jnp.broadcast_to(tot3[k], (LANES,)))
                            if s < S - 1:
                                e = e * jnp.broadcast_to(vv[s], (LANES,))
                            row = rows3[k]
                            accs = [accs[j] + e * row[j] for j in range(NJ)]
                            denom = denom + e
                    rv = 1.0 / denom
                    for j in range(NJ):
                        outb_v[b, hh, pl.ds(j * LANES, LANES)] = accs[j] * rv

                def pool_one(hh, _c):
                    pool_hop(hh, _c)
                    return _c

                lax.fori_loop(0, HCH, pool_one, c)
                outcp(c, b).start()

                @pl.when(g < NCH // NBUF - 1)
                def _():
                    gather(c + NBUF, b).start()
            return 0

        lax.fori_loop(0, NCH // NBUF, group, 0)
        for b in range(NBUF):
            outcp(NCH - NBUF + b, b).wait()

    return k(hf, qs, pi_t, po, val_t)


BH = 1024


def _mlp_body(hop_ref, W1_ref, b1_ref, W2g_ref, g2_ref, c2_ref, out_ref):
    x = hop_ref[...]
    h = jnp.dot(x.astype(jnp.bfloat16), W1_ref[...],
                preferred_element_type=jnp.float32)
    h = jnp.maximum(h + b1_ref[...], 0.0)
    mu = jnp.mean(h, axis=-1, keepdims=True)
    m2 = jnp.mean(h * h, axis=-1, keepdims=True)
    rs = lax.rsqrt(m2 - mu * mu + 1e-5)
    z = jnp.dot(h.astype(jnp.bfloat16), W2g_ref[...],
                preferred_element_type=jnp.float32)
    res = rs * z - (rs * mu) * g2_ref[...] + c2_ref[...]
    out_ref[...] = res.T


def _tc_mlp(hop_hf, W1, b1, W2g, g2, c2):
    nh = hop_hf.shape[0]
    return pl.pallas_call(
        _mlp_body,
        grid=(nh // BH,),
        in_specs=[
            pl.BlockSpec((BH, D), lambda i: (i, 0)),
            pl.BlockSpec((D, HID), lambda i: (0, 0)),
            pl.BlockSpec((1, HID), lambda i: (0, 0)),
            pl.BlockSpec((HID, TT), lambda i: (0, 0)),
            pl.BlockSpec((1, TT), lambda i: (0, 0)),
            pl.BlockSpec((1, TT), lambda i: (0, 0)),
        ],
        out_specs=pl.BlockSpec((TT, BH), lambda i: (0, i)),
        out_shape=jax.ShapeDtypeStruct((TT, nh), jnp.float32),
    )(hop_hf, W1, b1, W2g, g2, c2)


def kernel(hf, q, W1, b1, gamma, beta, W2, b2, all_hop_pi, all_hop_pi_stats,
           all_hop_po):
    pi_t = all_hop_pi.T
    val_t = (all_hop_pi_stats != -1).astype(jnp.float32).T

    qs = q * (1.0 / math.sqrt(D))
    W2g = (gamma[:, None] * W2).astype(jnp.bfloat16)
    g2 = (gamma @ W2).reshape(1, TT)
    c2 = (beta @ W2 + b2).reshape(1, TT)
    W1b = W1.astype(jnp.bfloat16)
    b1r = b1.reshape(1, HID)
    nh = H // NSPLIT
    outs = []
    for i in range(NSPLIT):
        hop_hf = _sc_pool(hf, qs, pi_t, all_hop_po, val_t, i * nh, nh)
        outs.append(_tc_mlp(hop_hf, W1b, b1r, W2g, g2, c2))
    return jnp.concatenate(outs, axis=1).T
